# Initial kernel scaffold; baseline (speedup 1.0000x reference)
import jax
import jax.numpy as jnp
from jax.experimental import pallas as pl


def kernel(x, centroids):
    raise NotImplementedError("write your pallas kernel here")


# speedup vs baseline: 1.6687x; 1.6687x over previous
import functools

import jax
import jax.numpy as jnp
from jax import lax
from jax.experimental import pallas as pl
from jax.experimental.pallas import tpu as pltpu
from jax.experimental.pallas import tpu_sc as plsc

N = 16384
K = 1024
D = 64

BN = 512
NB = N // BN


def _assign_body(x_ref, c_ref, assign_ref, sumd_ref):
    i = pl.program_id(0)
    x = x_ref[...]
    c = c_ref[...]
    x2 = jnp.sum(x * x, axis=1, keepdims=True)
    c2 = jnp.sum(c * c, axis=1)
    xc = lax.dot_general(x, c, (((1,), (1,)), ((), ())),
                         preferred_element_type=jnp.float32)
    dist = x2 - 2.0 * xc + c2[None, :]
    minv = jnp.min(dist, axis=1, keepdims=True)
    ids = lax.broadcasted_iota(jnp.int32, (BN, K), 1)
    cand = jnp.where(dist == minv, ids, K)
    assign_ref[0, 0, :] = jnp.min(cand, axis=1)

    @pl.when(i == 0)
    def _():
        sumd_ref[...] = jnp.zeros((1, 1), jnp.float32)

    sumd_ref[...] += jnp.sum(minv).reshape(1, 1)


def _assign_stage(x, centroids):
    assign3, sumd = pl.pallas_call(
        _assign_body,
        grid=(NB,),
        in_specs=[
            pl.BlockSpec((BN, D), lambda i: (i, 0)),
            pl.BlockSpec((K, D), lambda i: (0, 0)),
        ],
        out_specs=[
            pl.BlockSpec((1, 1, BN), lambda i: (i, 0, 0)),
            pl.BlockSpec((1, 1), lambda i: (0, 0)),
        ],
        out_shape=[
            jax.ShapeDtypeStruct((NB, 1, BN), jnp.int32),
            jax.ShapeDtypeStruct((1, 1), jnp.float32),
        ],
    )(x, centroids)
    return assign3.reshape(N), sumd[0, 0]


CHUNK = 128
PTS_PER_TILE = N // 16
ROWS_PER_TILE = K // 16


def _sc_body(assign_hbm, x_hbm, out_hbm,
             idx_v, x_v, ones_v, row_v, cnt_v, sums_sh, cnts_sh):
    cid = lax.axis_index("c")
    sid = lax.axis_index("s")

    @pl.when(cid == 0)
    def _():
        zero16 = jnp.zeros((16,), jnp.float32)
        one16 = jnp.ones((16,), jnp.float32)

        def fill_const(r, _):
            for cc in range(D // 16):
                row_v[r, pl.ds(cc * 16, 16)] = zero16
            cnt_v[r, :] = zero16
            return 0

        lax.fori_loop(0, ROWS_PER_TILE, fill_const, 0)

        def fill_ones(r, _):
            ones_v[r, :] = one16
            return 0

        lax.fori_loop(0, CHUNK, fill_ones, 0)

        rbase = sid * ROWS_PER_TILE
        pltpu.sync_copy(row_v, sums_sh.at[pl.ds(rbase, ROWS_PER_TILE)])
        pltpu.sync_copy(cnt_v, cnts_sh.at[pl.ds(rbase, ROWS_PER_TILE)])
        plsc.subcore_barrier()

        pbase = sid * PTS_PER_TILE

        def chunk_step(j, _):
            b = pbase + j * CHUNK
            pltpu.sync_copy(assign_hbm.at[pl.ds(b, CHUNK)], idx_v)
            pltpu.sync_copy(x_hbm.at[pl.ds(b, CHUNK)], x_v)
            pltpu.sync_copy(x_v, sums_sh.at[idx_v], add=True)
            pltpu.sync_copy(ones_v, cnts_sh.at[idx_v], add=True)
            return 0

        lax.fori_loop(0, PTS_PER_TILE // CHUNK, chunk_step, 0)
        plsc.subcore_barrier()

        pltpu.sync_copy(sums_sh.at[pl.ds(rbase, ROWS_PER_TILE)], row_v)
        pltpu.sync_copy(cnts_sh.at[pl.ds(rbase, ROWS_PER_TILE)], cnt_v)

        def div_row(r, _):
            cnt = cnt_v[r, :]
            for cc in range(D // 16):
                sl = pl.ds(cc * 16, 16)
                row_v[r, sl] = row_v[r, sl] / cnt
            return 0

        lax.fori_loop(0, ROWS_PER_TILE, div_row, 0)
        pltpu.sync_copy(row_v, out_hbm.at[pl.ds(rbase, ROWS_PER_TILE)])


def _update_stage(assignments, x):
    mesh = plsc.VectorSubcoreMesh(core_axis_name="c", subcore_axis_name="s")
    return pl.kernel(
        _sc_body,
        out_type=jax.ShapeDtypeStruct((K, D), jnp.float32),
        mesh=mesh,
        scratch_types=[
            pltpu.VMEM((CHUNK,), jnp.int32),
            pltpu.VMEM((CHUNK, D), jnp.float32),
            pltpu.VMEM((CHUNK, 16), jnp.float32),
            pltpu.VMEM((ROWS_PER_TILE, D), jnp.float32),
            pltpu.VMEM((ROWS_PER_TILE, 16), jnp.float32),
            pltpu.VMEM_SHARED((K, D), jnp.float32),
            pltpu.VMEM_SHARED((K, 16), jnp.float32),
        ],
    )(assignments, x)


def kernel(x, centroids):
    assignments, sum_distances = _assign_stage(x, centroids)
    updated_centroids = _update_stage(assignments, x)
    return assignments, updated_centroids, sum_distances
